# baseline (speedup 1.0000x reference)
import functools

import jax
import jax.numpy as jnp
from jax import lax
from jax.experimental import pallas as pl
from jax.experimental.pallas import tpu as pltpu
from jax.experimental.pallas import tpu_sc as plsc

NC = 2
NS = 16
NW = NC * NS
C = 128


def _patch_rows(img):
    h, w, f = img.shape
    r1 = jnp.roll(img, -1, axis=1)
    r2 = jnp.roll(img, -1, axis=0)
    r3 = jnp.roll(r2, -1, axis=1)
    return jnp.concatenate([img, r1, r2, r3], axis=-1).reshape(h * w, 4 * f)


def _build_pairs(spatial_emb, temporal_emb):
    res = [spatial_emb[i].shape[-1] for i in range(len(spatial_emb))]
    parts = []
    pairs = []
    off = 0
    sel = [(0, 1), (0, 2), (1, 2)]
    for g in range(3):
        a, b = sel[g]
        for l in range(len(res)):
            r = res[l]
            img = jnp.transpose(spatial_emb[l][g], (1, 2, 0))
            parts.append(_patch_rows(img))
            pairs.append(dict(a=a, b=b, sx=float(r), cx=-0.5,
                              sy=float(r), cy=-0.5, w=r, h=r,
                              off=off, col=g * 8 + l * 2))
            off += r * r
    for g in range(3):
        for l in range(len(res)):
            r = res[l]
            img = jnp.transpose(temporal_emb[l][g], (1, 2, 0))
            dim_t = img.shape[1]
            parts.append(_patch_rows(img))
            pairs.append(dict(a=3, b=g, sx=float(dim_t - 1), cx=0.0,
                              sy=float(r - 1), cy=0.0, w=dim_t, h=r,
                              off=off, col=(3 + g) * 8 + l * 2))
            off += r * dim_t
    table = jnp.concatenate(parts, axis=0)
    return table, pairs


def _sc_kernel(pairs, n_points, coords, table):
    npairs = len(pairs)
    ppw = n_points // NW
    nch = ppw // C
    nf = 2 * npairs

    mesh = plsc.VectorSubcoreMesh(core_axis_name="c", subcore_axis_name="s")

    @functools.partial(
        pl.kernel,
        out_type=jax.ShapeDtypeStruct((n_points, nf), jnp.float32),
        mesh=mesh,
        compiler_params=pltpu.CompilerParams(needs_layout_passes=False, use_tc_tiling_on_sc=False),
        scratch_types=[
            pltpu.VMEM((4, C), jnp.float32),
            pltpu.VMEM((npairs, C), jnp.int32),
            pltpu.VMEM((npairs, C), jnp.float32),
            pltpu.VMEM((npairs, C), jnp.float32),
            [pltpu.VMEM((C, 8), jnp.float32) for _ in range(npairs)],
            pltpu.VMEM((C, nf), jnp.float32),
            pltpu.SemaphoreType.DMA,
            pltpu.SemaphoreType.DMA,
        ],
    )
    def run(coords_hbm, table_hbm, out_hbm,
            coords_v, idx_v, wx_v, wy_v, gath_v, out_v, csem, gsem):
        wid = lax.axis_index("s") * NC + lax.axis_index("c")
        base0 = wid * ppw

        def chunk_body(ch, carry):
            base = base0 + ch * C
            cds = [pltpu.async_copy(coords_hbm.at[k, pl.ds(base, C)],
                                    coords_v.at[k], csem)
                   for k in range(4)]
            for d in cds:
                d.wait()

            descs = []
            for p, prm in enumerate(pairs):
                a_row, b_row = prm["a"], prm["b"]
                sx, cx = prm["sx"], prm["cx"]
                sy, cy = prm["sy"], prm["cy"]
                w, h, off = prm["w"], prm["h"], prm["off"]

                def istep(i, c2, a_row=a_row, b_row=b_row, sx=sx, cx=cx,
                          sy=sy, cy=cy, w=w, h=h, off=off, p=p):
                    sl = pl.ds(i * 16, 16)
                    av = coords_v[a_row, sl]
                    bv = coords_v[b_row, sl]
                    ix = av * sx + cx
                    iy = bv * sy + cy
                    ui = jnp.clip(ix.astype(jnp.int32), 0, w - 2)
                    vi = jnp.clip(iy.astype(jnp.int32), 0, h - 2)
                    wx = jnp.clip(ix - ui.astype(jnp.float32), 0.0, 1.0)
                    wy = jnp.clip(iy - vi.astype(jnp.float32), 0.0, 1.0)
                    idx_v[p, sl] = vi * w + ui + off
                    wx_v[p, sl] = wx
                    wy_v[p, sl] = wy
                    return c2

                lax.fori_loop(0, C // 16, istep, 0)
                descs.append(pltpu.async_copy(
                    table_hbm.at[idx_v.at[p]], gath_v[p], gsem))

            for p, prm in enumerate(pairs):
                descs[p].wait()
                col = prm["col"]

                def bstep(i, c2, p=p, col=col):
                    sl = pl.ds(i * 16, 16)
                    rows = lax.iota(jnp.int32, 16) + i * 16
                    wx1 = wx_v[p, sl]
                    wy1 = wy_v[p, sl]
                    wx0 = 1.0 - wx1
                    wy0 = 1.0 - wy1
                    w00 = wx0 * wy0
                    w01 = wx1 * wy0
                    w10 = wx0 * wy1
                    w11 = wx1 * wy1
                    g = gath_v[p]
                    cs = [plsc.load_gather(
                        g, [rows, jnp.full((16,), k, jnp.int32)])
                        for k in range(8)]
                    f0 = w00 * cs[0] + w01 * cs[2] + w10 * cs[4] + w11 * cs[6]
                    f1 = w00 * cs[1] + w01 * cs[3] + w10 * cs[5] + w11 * cs[7]
                    ccol = jnp.full((16,), col, jnp.int32)
                    plsc.store_scatter(out_v, [rows, ccol], f0)
                    plsc.store_scatter(out_v, [rows, ccol + 1], f1)
                    return c2

                lax.fori_loop(0, C // 16, bstep, 0)

            pltpu.sync_copy(out_v, out_hbm.at[pl.ds(base, C)])
            return carry

        lax.fori_loop(0, nch, chunk_body, 0)

    return run(coords, table)


def kernel(xyz, t, batch, spatial_emb, temporal_emb, bounds):
    bash = xyz.shape
    xyz = xyz.reshape(-1, xyz.shape[-1])
    t = t.reshape(-1, t.shape[-1])
    n = xyz.shape[0]
    xyzn = (xyz - bounds[0]) / (bounds[1] - bounds[0])
    coords = jnp.concatenate([xyzn.T, t[:, :1].T], axis=0)
    table, pairs = _build_pairs(spatial_emb, temporal_emb)
    out = _sc_kernel(pairs, n, coords, table)
    return out.reshape(*bash[:-1], out.shape[-1])


# speedup vs baseline: 1.0040x; 1.0040x over previous
import functools

import jax
import jax.numpy as jnp
from jax import lax
from jax.experimental import pallas as pl
from jax.experimental.pallas import tpu as pltpu
from jax.experimental.pallas import tpu_sc as plsc

NC = 2
NS = 16
NW = NC * NS
C = 128


def _patch_rows(img):
    f, h, w = img.shape
    r1 = jnp.roll(img, -1, axis=2)
    r2 = jnp.roll(img, -1, axis=1)
    r3 = jnp.roll(r2, -1, axis=2)
    quad = jnp.stack([img, r1, r2, r3], axis=-1)
    return jnp.concatenate([quad[i] for i in range(f)],
                           axis=-1).reshape(h * w, f * 4)


def _build_pairs(spatial_emb, temporal_emb):
    res = [spatial_emb[i].shape[-1] for i in range(len(spatial_emb))]
    parts = []
    pairs = []
    off = 0
    sel = [(0, 1), (0, 2), (1, 2)]
    for g in range(3):
        a, b = sel[g]
        for l in range(len(res)):
            r = res[l]
            img = spatial_emb[l][g]
            parts.append(_patch_rows(img))
            pairs.append(dict(a=a, b=b, sx=float(r), cx=-0.5,
                              sy=float(r), cy=-0.5, w=r, h=r,
                              off=off, col=g * 8 + l * 2))
            off += r * r
    for g in range(3):
        for l in range(len(res)):
            r = res[l]
            img = temporal_emb[l][g]
            dim_t = img.shape[2]
            parts.append(_patch_rows(img))
            pairs.append(dict(a=3, b=g, sx=float(dim_t - 1), cx=0.0,
                              sy=float(r - 1), cy=0.0, w=dim_t, h=r,
                              off=off, col=(3 + g) * 8 + l * 2))
            off += r * dim_t
    table = jnp.concatenate(parts, axis=0)
    return table, pairs


def _sc_kernel(pairs, n_points, coords, table):
    npairs = len(pairs)
    ppw = n_points // NW
    nch = ppw // C
    nf = 2 * npairs

    mesh = plsc.VectorSubcoreMesh(core_axis_name="c", subcore_axis_name="s")

    @functools.partial(
        pl.kernel,
        out_type=jax.ShapeDtypeStruct((n_points // C, nf, C), jnp.float32),
        mesh=mesh,
        compiler_params=pltpu.CompilerParams(needs_layout_passes=False, use_tc_tiling_on_sc=False),
        scratch_types=[
            pltpu.VMEM((4, C), jnp.float32),
            pltpu.VMEM((npairs, C), jnp.int32),
            pltpu.VMEM((npairs, C), jnp.float32),
            pltpu.VMEM((npairs, C), jnp.float32),
            [pltpu.VMEM((C, 8), jnp.float32) for _ in range(npairs)],
            pltpu.VMEM((nf, C), jnp.float32),
            pltpu.SemaphoreType.DMA,
            pltpu.SemaphoreType.DMA,
        ],
    )
    def run(coords_hbm, table_hbm, out_hbm,
            coords_v, idx_v, wx_v, wy_v, gath_v, out_v, csem, gsem):
        wid = lax.axis_index("s") * NC + lax.axis_index("c")
        base0 = wid * ppw

        def chunk_body(ch, carry):
            base = base0 + ch * C
            cid = wid * nch + ch
            cds = [pltpu.async_copy(coords_hbm.at[k, pl.ds(base, C)],
                                    coords_v.at[k], csem)
                   for k in range(4)]
            for d in cds:
                d.wait()

            descs = []
            for p, prm in enumerate(pairs):
                a_row, b_row = prm["a"], prm["b"]
                sx, cx = prm["sx"], prm["cx"]
                sy, cy = prm["sy"], prm["cy"]
                w, h, off = prm["w"], prm["h"], prm["off"]

                def istep(i, c2, a_row=a_row, b_row=b_row, sx=sx, cx=cx,
                          sy=sy, cy=cy, w=w, h=h, off=off, p=p):
                    sl = pl.ds(i * 16, 16)
                    av = coords_v[a_row, sl]
                    bv = coords_v[b_row, sl]
                    ix = av * sx + cx
                    iy = bv * sy + cy
                    ui = jnp.clip(ix.astype(jnp.int32), 0, w - 2)
                    vi = jnp.clip(iy.astype(jnp.int32), 0, h - 2)
                    wx = jnp.clip(ix - ui.astype(jnp.float32), 0.0, 1.0)
                    wy = jnp.clip(iy - vi.astype(jnp.float32), 0.0, 1.0)
                    idx_v[p, sl] = vi * w + ui + off
                    wx_v[p, sl] = wx
                    wy_v[p, sl] = wy
                    return c2

                lax.fori_loop(0, C // 16, istep, 0)
                descs.append(pltpu.async_copy(
                    table_hbm.at[idx_v.at[p]], gath_v[p], gsem))

            for p, prm in enumerate(pairs):
                descs[p].wait()
                col = prm["col"]

                def bstep(i, c2, p=p, col=col):
                    sl = pl.ds(i * 16, 16)
                    rows = lax.iota(jnp.int32, 16) + i * 16
                    wx1 = wx_v[p, sl]
                    wy1 = wy_v[p, sl]
                    wx0 = 1.0 - wx1
                    wy0 = 1.0 - wy1
                    w00 = wx0 * wy0
                    w01 = wx1 * wy0
                    w10 = wx0 * wy1
                    w11 = wx1 * wy1
                    g = gath_v[p]
                    cs = [plsc.load_gather(
                        g, [rows, jnp.full((16,), k, jnp.int32)])
                        for k in range(8)]
                    f0 = w00 * cs[0] + w01 * cs[1] + w10 * cs[2] + w11 * cs[3]
                    f1 = w00 * cs[4] + w01 * cs[5] + w10 * cs[6] + w11 * cs[7]
                    out_v[col, sl] = f0
                    out_v[col + 1, sl] = f1
                    return c2

                lax.fori_loop(0, C // 16, bstep, 0)

            pltpu.sync_copy(out_v, out_hbm.at[cid])
            return carry

        lax.fori_loop(0, nch, chunk_body, 0)

    return run(coords, table)


def kernel(xyz, t, batch, spatial_emb, temporal_emb, bounds):
    bash = xyz.shape
    xyz = xyz.reshape(-1, xyz.shape[-1])
    t = t.reshape(-1, t.shape[-1])
    n = xyz.shape[0]
    xyzn = (xyz - bounds[0]) / (bounds[1] - bounds[0])
    coords = jnp.concatenate([xyzn.T, t[:, :1].T], axis=0)
    table, pairs = _build_pairs(spatial_emb, temporal_emb)
    out = _sc_kernel(pairs, n, coords, table)
    out = out.transpose(0, 2, 1).reshape(n, -1)
    return out.reshape(*bash[:-1], out.shape[-1])


# speedup vs baseline: 2.8316x; 2.8203x over previous
import functools

import jax
import jax.numpy as jnp
from jax import lax
from jax.experimental import pallas as pl
from jax.experimental.pallas import tpu as pltpu
from jax.experimental.pallas import tpu_sc as plsc

NC = 2
NS = 16
NW = NC * NS
C = 128


def _meta(spatial_emb, temporal_emb):
    res = [spatial_emb[i].shape[-1] for i in range(len(spatial_emb))]
    dim_t = temporal_emb[0].shape[-1]
    pairs = []
    images = []
    off = 0
    sel = [(0, 1), (0, 2), (1, 2)]
    for g in range(3):
        a, b = sel[g]
        for l in range(len(res)):
            r = res[l]
            pairs.append(dict(a=a, b=b, sx=float(r), cx=-0.5,
                              sy=float(r), cy=-0.5, w=r, h=r,
                              off=off, col=g * 8 + l * 2))
            images.append(dict(ref=l, g=g, h=r, w=r, ws=r, off=off))
            off += r * r
    for g in range(3):
        for l in range(len(res)):
            r = res[l]
            pairs.append(dict(a=3, b=g, sx=float(dim_t - 1), cx=0.0,
                              sy=float(r - 1), cy=0.0, w=dim_t, h=r,
                              off=off, col=(3 + g) * 8 + l * 2))
            images.append(dict(ref=4 + l, g=g, h=r, w=dim_t,
                               ws=-(-dim_t // 8) * 8, off=off))
            off += r * dim_t
    return pairs, images, off


def _sc_build_table(spatial_emb, temporal_emb, images, tot_rows):
    mesh = plsc.VectorSubcoreMesh(core_axis_name="c", subcore_axis_name="s")
    slab_h, slab_w = 40, 1040
    ost_n = 8336

    @functools.partial(
        pl.kernel,
        out_type=jax.ShapeDtypeStruct((tot_rows * 8,), jnp.float32),
        mesh=mesh,
        compiler_params=pltpu.CompilerParams(needs_layout_passes=False,
                                             use_tc_tiling_on_sc=False),
        scratch_types=[
            pltpu.VMEM((slab_h, slab_w), jnp.float32),
            pltpu.VMEM((slab_h, slab_w), jnp.float32),
            pltpu.VMEM((ost_n,), jnp.float32),
            pltpu.SemaphoreType.DMA,
        ],
    )
    def build(s0, s1, s2, s3, t0, t1, t2, t3, table_out, fsl, gsl, ost, isem):
        wid = lax.axis_index("s") * NC + lax.axis_index("c")
        srcs = [s0, s1, s2, s3, t0, t1, t2, t3]
        for im in images:
            src = srcs[im["ref"]]
            g, h, w, off = im["g"], im["h"], im["w"], im["off"]
            ws = im["ws"]
            nb = -(-(h - 1) // NW)
            rows = min(-(-(nb + 1) // 8) * 8, h)
            v0 = jnp.minimum(wid * nb, h - 1)
            v1 = jnp.minimum(v0 + nb, h - 1)
            start = jnp.minimum(v0, h - rows)
            d1 = pltpu.async_copy(src.at[g, 0, pl.ds(start, rows), :],
                                  fsl.at[pl.ds(0, rows), pl.ds(0, ws)], isem)
            d2 = pltpu.async_copy(src.at[g, 1, pl.ds(start, rows), :],
                                  gsl.at[pl.ds(0, rows), pl.ds(0, ws)], isem)
            d1.wait()
            d2.wait()
            njj = -(-w // 16)

            def row_body(v, cr, g=g, h=h, w=w, off=off, njj=njj, start=start):
                vr = v - start

                def jbody(jj, c2, vr=vr, w=w):
                    u = jj * 16
                    lanes = lax.iota(jnp.int32, 16)
                    base = (lanes + u) * 8
                    a0 = fsl[vr, pl.ds(u, 16)]
                    a1 = fsl[vr, pl.ds(u + 1, 16)]
                    b0 = gsl[vr, pl.ds(u, 16)]
                    b1 = gsl[vr, pl.ds(u + 1, 16)]
                    c0 = fsl[vr + 1, pl.ds(u, 16)]
                    c1 = fsl[vr + 1, pl.ds(u + 1, 16)]
                    e0 = gsl[vr + 1, pl.ds(u, 16)]
                    e1 = gsl[vr + 1, pl.ds(u + 1, 16)]
                    plsc.store_scatter(ost, [base], a0)
                    plsc.store_scatter(ost, [base + 1], b0)
                    plsc.store_scatter(ost, [base + 2], a1)
                    plsc.store_scatter(ost, [base + 3], b1)
                    plsc.store_scatter(ost, [base + 4], c0)
                    plsc.store_scatter(ost, [base + 5], e0)
                    plsc.store_scatter(ost, [base + 6], c1)
                    plsc.store_scatter(ost, [base + 7], e1)
                    return c2

                lax.fori_loop(0, njj, jbody, 0)
                pltpu.sync_copy(
                    ost.at[pl.ds(0, w * 8)],
                    table_out.at[pl.ds((off + v * w) * 8, w * 8)])
                return cr

            lax.fori_loop(v0, v1, row_body, 0)

    return build(*spatial_emb, *temporal_emb)


def _sc_kernel(pairs, n_points, coords, table):
    npairs = len(pairs)
    ppw = n_points // NW
    nch = ppw // C
    nf = 2 * npairs

    mesh = plsc.VectorSubcoreMesh(core_axis_name="c", subcore_axis_name="s")

    @functools.partial(
        pl.kernel,
        out_type=jax.ShapeDtypeStruct((n_points // C, nf, C), jnp.float32),
        mesh=mesh,
        compiler_params=pltpu.CompilerParams(needs_layout_passes=False, use_tc_tiling_on_sc=False),
        scratch_types=[
            pltpu.VMEM((4, C), jnp.float32),
            pltpu.VMEM((npairs, C), jnp.int32),
            pltpu.VMEM((npairs, C), jnp.float32),
            pltpu.VMEM((npairs, C), jnp.float32),
            [pltpu.VMEM((C, 8), jnp.float32) for _ in range(npairs)],
            pltpu.VMEM((nf, C), jnp.float32),
            pltpu.SemaphoreType.DMA,
            pltpu.SemaphoreType.DMA,
        ],
    )
    def run(coords_hbm, table_hbm, out_hbm,
            coords_v, idx_v, wx_v, wy_v, gath_v, out_v, csem, gsem):
        wid = lax.axis_index("s") * NC + lax.axis_index("c")
        base0 = wid * ppw

        def chunk_body(ch, carry):
            base = base0 + ch * C
            cid = wid * nch + ch
            cds = [pltpu.async_copy(coords_hbm.at[k, pl.ds(base, C)],
                                    coords_v.at[k], csem)
                   for k in range(4)]
            for d in cds:
                d.wait()

            descs = []
            for p, prm in enumerate(pairs):
                a_row, b_row = prm["a"], prm["b"]
                sx, cx = prm["sx"], prm["cx"]
                sy, cy = prm["sy"], prm["cy"]
                w, h, off = prm["w"], prm["h"], prm["off"]

                def istep(i, c2, a_row=a_row, b_row=b_row, sx=sx, cx=cx,
                          sy=sy, cy=cy, w=w, h=h, off=off, p=p):
                    sl = pl.ds(i * 16, 16)
                    av = coords_v[a_row, sl]
                    bv = coords_v[b_row, sl]
                    ix = av * sx + cx
                    iy = bv * sy + cy
                    ui = jnp.clip(ix.astype(jnp.int32), 0, w - 2)
                    vi = jnp.clip(iy.astype(jnp.int32), 0, h - 2)
                    wx = jnp.clip(ix - ui.astype(jnp.float32), 0.0, 1.0)
                    wy = jnp.clip(iy - vi.astype(jnp.float32), 0.0, 1.0)
                    idx_v[p, sl] = vi * w + ui + off
                    wx_v[p, sl] = wx
                    wy_v[p, sl] = wy
                    return c2

                lax.fori_loop(0, C // 16, istep, 0)
                descs.append(pltpu.async_copy(
                    table_hbm.at[idx_v.at[p]], gath_v[p], gsem))

            for p, prm in enumerate(pairs):
                descs[p].wait()
                col = prm["col"]

                def bstep(i, c2, p=p, col=col):
                    sl = pl.ds(i * 16, 16)
                    rows = lax.iota(jnp.int32, 16) + i * 16
                    wx1 = wx_v[p, sl]
                    wy1 = wy_v[p, sl]
                    wx0 = 1.0 - wx1
                    wy0 = 1.0 - wy1
                    w00 = wx0 * wy0
                    w01 = wx1 * wy0
                    w10 = wx0 * wy1
                    w11 = wx1 * wy1
                    g = gath_v[p]
                    cs = [plsc.load_gather(
                        g, [rows, jnp.full((16,), k, jnp.int32)])
                        for k in range(8)]
                    f0 = w00 * cs[0] + w01 * cs[2] + w10 * cs[4] + w11 * cs[6]
                    f1 = w00 * cs[1] + w01 * cs[3] + w10 * cs[5] + w11 * cs[7]
                    out_v[col, sl] = f0
                    out_v[col + 1, sl] = f1
                    return c2

                lax.fori_loop(0, C // 16, bstep, 0)

            pltpu.sync_copy(out_v, out_hbm.at[cid])
            return carry

        lax.fori_loop(0, nch, chunk_body, 0)

    return run(coords, table)


def kernel(xyz, t, batch, spatial_emb, temporal_emb, bounds):
    bash = xyz.shape
    xyz = xyz.reshape(-1, xyz.shape[-1])
    t = t.reshape(-1, t.shape[-1])
    n = xyz.shape[0]
    xyzn = (xyz - bounds[0]) / (bounds[1] - bounds[0])
    coords = jnp.concatenate([xyzn.T, t[:, :1].T], axis=0)
    pairs, images, tot_rows = _meta(spatial_emb, temporal_emb)
    dim_t = temporal_emb[0].shape[-1]
    tpad = -(-dim_t // 8) * 8 - dim_t
    temporal_padded = tuple(
        jnp.pad(te, ((0, 0), (0, 0), (0, 0), (0, tpad))) if tpad else te
        for te in temporal_emb)
    table = _sc_build_table(spatial_emb, temporal_padded,
                            images, tot_rows).reshape(tot_rows, 8)
    out = _sc_kernel(pairs, n, coords, table)
    out = out.transpose(0, 2, 1).reshape(n, -1)
    return out.reshape(*bash[:-1], out.shape[-1])
